# baseline (speedup 1.0000x reference)
import functools

import jax
import jax.numpy as jnp
from jax import lax
from jax.experimental import pallas as pl
from jax.experimental.pallas import tpu as pltpu
from jax.experimental.pallas import tpu_sc as plsc

N_CELLS = 1024
N_GENES = 500
N_GENES_PAD = 504
N_EMB = 100
E_TC = 95
N_E_SC = N_EMB - E_TC
BLOCK_E = 5

_NC = 2
_ROWS_W = 16
_CB = 128


def _make_sc_part():
    mesh = plsc.VectorSubcoreMesh(core_axis_name="c", subcore_axis_name="s")
    n_full = N_GENES // _ROWS_W
    g_tail = N_GENES - n_full * _ROWS_W

    @functools.partial(
        pl.kernel,
        mesh=mesh,
        out_type=(
            jax.ShapeDtypeStruct((N_GENES,), jnp.float32),
            jax.ShapeDtypeStruct((N_GENES_PAD, N_CELLS), jnp.float32),
        ),
        scratch_types=[
            pltpu.VMEM((_ROWS_W,), jnp.int32),
            pltpu.VMEM((_ROWS_W,), jnp.float32),
            pltpu.VMEM((N_EMB,), jnp.float32),
        ]
        + [pltpu.VMEM((_ROWS_W, _CB), jnp.float32) for _ in range(N_E_SC)]
        + [
            pltpu.VMEM((_ROWS_W, _CB), jnp.float32),
            pltpu.SemaphoreType.DMA,
            pltpu.SemaphoreType.DMA,
        ],
        compiler_params=pltpu.CompilerParams(use_tc_tiling_on_sc=True, needs_layout_passes=False),
    )
    def sc_part(table_hbm, idx_hbm, xt_hbm, w_hbm, bias_out, part_out,
                idx_v, rows_v, w_v, *bufs_acc_sems):
        bufs = bufs_acc_sems[:N_E_SC]
        acc = bufs_acc_sems[N_E_SC]
        sem_g = bufs_acc_sems[N_E_SC + 1]
        sem_s = bufs_acc_sems[N_E_SC + 2]
        wid = lax.axis_index("s") * _NC + lax.axis_index("c")

        @pl.when(wid < n_full)
        def _full():
            base = wid * _ROWS_W
            pltpu.sync_copy(idx_hbm.at[pl.ds(base, _ROWS_W)], idx_v)
            pltpu.async_copy(table_hbm.at[idx_v], rows_v, sem_g).wait()
            pltpu.sync_copy(rows_v, bias_out.at[pl.ds(base, _ROWS_W)])

        @pl.when(wid == n_full)
        def _tail():
            base = n_full * _ROWS_W
            pltpu.sync_copy(idx_hbm.at[pl.ds(base, g_tail)], idx_v.at[pl.ds(0, g_tail)])
            pltpu.async_copy(
                table_hbm.at[idx_v.at[pl.ds(0, g_tail)]], rows_v.at[pl.ds(0, g_tail)], sem_g
            ).wait()
            pltpu.sync_copy(rows_v.at[pl.ds(0, g_tail)], bias_out.at[pl.ds(base, g_tail)])

        rbase = jnp.minimum(wid * _ROWS_W, N_GENES_PAD - _ROWS_W)
        pltpu.sync_copy(w_hbm, w_v)
        splats = [
            plsc.load_gather(w_v, [jnp.full((16,), E_TC + j, jnp.int32)])
            for j in range(N_E_SC)
        ]
        for cb in range(N_CELLS // _CB):
            cps = [
                pltpu.async_copy(
                    xt_hbm.at[E_TC + j, pl.ds(rbase, _ROWS_W), pl.ds(cb * _CB, _CB)],
                    bufs[j],
                    sem_s,
                )
                for j in range(N_E_SC)
            ]
            for cp in cps:
                cp.wait()

            def chunk(i, carry):
                r = i // (_CB // 16)
                lo = (i % (_CB // 16)) * 16
                v = splats[0] * bufs[0][r, pl.ds(lo, 16)]
                for j in range(1, N_E_SC):
                    v += splats[j] * bufs[j][r, pl.ds(lo, 16)]
                acc[r, pl.ds(lo, 16)] = v
                return carry

            lax.fori_loop(0, _ROWS_W * (_CB // 16), chunk, 0)
            pltpu.sync_copy(acc, part_out.at[pl.ds(rbase, _ROWS_W), pl.ds(cb * _CB, _CB)])

    return sc_part


_sc_part = _make_sc_part()


def _matvec_body(x_ref, w_ref, o_ref):
    i = pl.program_id(0)
    s = x_ref[0] * w_ref[i * BLOCK_E, 0]
    for k in range(1, BLOCK_E):
        s += x_ref[k] * w_ref[i * BLOCK_E + k, 0]

    @pl.when(i == 0)
    def _init():
        o_ref[...] = s

    @pl.when(i > 0)
    def _acc():
        o_ref[...] += s


def _tc_matvec(xt, w2):
    grid = (E_TC // BLOCK_E,)
    return pl.pallas_call(
        _matvec_body,
        grid=grid,
        in_specs=[
            pl.BlockSpec((BLOCK_E, N_GENES, N_CELLS), lambda i: (i, 0, 0)),
            pl.BlockSpec(memory_space=pltpu.SMEM),
        ],
        out_specs=pl.BlockSpec((N_GENES, N_CELLS), lambda i: (0, 0)),
        out_shape=jax.ShapeDtypeStruct((N_GENES, N_CELLS), jnp.float32),
    )(xt, w2)


def _combine_body(p_ref, q_ref, b_ref, o_ref):
    b = b_ref[...].reshape(N_GENES, 1)
    o_ref[...] = p_ref[...] + q_ref[0:N_GENES, :] + b


def _tc_combine(partial_tc, partial_sc, bias_g):
    return pl.pallas_call(
        _combine_body,
        in_specs=[
            pl.BlockSpec((N_GENES, N_CELLS), lambda: (0, 0)),
            pl.BlockSpec((N_GENES_PAD, N_CELLS), lambda: (0, 0)),
            pl.BlockSpec((N_GENES,), lambda: (0,)),
        ],
        out_specs=pl.BlockSpec((N_GENES, N_CELLS), lambda: (0, 0)),
        out_shape=jax.ShapeDtypeStruct((N_GENES, N_CELLS), jnp.float32),
    )(partial_tc, partial_sc, bias_g)


def kernel(cell_gene_embedding, gene_ix, weight1, bias1):
    xt = jnp.transpose(cell_gene_embedding, (2, 1, 0))
    partial_tc = _tc_matvec(xt, weight1.reshape(N_EMB, 1))
    bias_g, partial_sc = _sc_part(bias1, gene_ix.astype(jnp.int32), xt, weight1)
    out_t = _tc_combine(partial_tc, partial_sc, bias_g)
    return out_t.T


# speedup vs baseline: 1.0323x; 1.0323x over previous
import functools

import jax
import jax.numpy as jnp
from jax import lax
from jax.experimental import pallas as pl
from jax.experimental.pallas import tpu as pltpu
from jax.experimental.pallas import tpu_sc as plsc

N_CELLS = 1024
N_GENES = 500
N_EMB = 100
BLOCK_E = 5


def _make_sc_gather():
    mesh = plsc.VectorSubcoreMesh(core_axis_name="c", subcore_axis_name="s",
                                  num_cores=1)
    per_w = 32
    n_full = N_GENES // per_w
    tail = N_GENES - n_full * per_w

    @functools.partial(
        pl.kernel,
        mesh=mesh,
        out_type=jax.ShapeDtypeStruct((N_GENES,), jnp.float32),
        scratch_types=[
            pltpu.VMEM((per_w,), jnp.int32),
            pltpu.VMEM((per_w,), jnp.float32),
            pltpu.SemaphoreType.DMA,
        ],
    )
    def gather_bias(table_hbm, idx_hbm, out_hbm, idx_v, rows_v, sem):
        wid = lax.axis_index("s")

        @pl.when(wid < n_full)
        def _full():
            base = wid * per_w
            pltpu.sync_copy(idx_hbm.at[pl.ds(base, per_w)], idx_v)
            pltpu.async_copy(table_hbm.at[idx_v], rows_v, sem).wait()
            pltpu.sync_copy(rows_v, out_hbm.at[pl.ds(base, per_w)])

        @pl.when(wid == n_full)
        def _tail():
            base = n_full * per_w
            pltpu.sync_copy(idx_hbm.at[pl.ds(base, tail)], idx_v.at[pl.ds(0, tail)])
            pltpu.async_copy(
                table_hbm.at[idx_v.at[pl.ds(0, tail)]], rows_v.at[pl.ds(0, tail)], sem
            ).wait()
            pltpu.sync_copy(rows_v.at[pl.ds(0, tail)], out_hbm.at[pl.ds(base, tail)])

    return gather_bias


_sc_gather = _make_sc_gather()


def _matvec_body(x_ref, w_ref, o_ref):
    i = pl.program_id(0)
    s = x_ref[0] * w_ref[i * BLOCK_E, 0]
    for k in range(1, BLOCK_E):
        s += x_ref[k] * w_ref[i * BLOCK_E + k, 0]

    @pl.when(i == 0)
    def _init():
        o_ref[...] = s

    @pl.when(i > 0)
    def _acc():
        o_ref[...] += s


def _tc_matvec(xt, w2):
    grid = (N_EMB // BLOCK_E,)
    return pl.pallas_call(
        _matvec_body,
        grid=grid,
        in_specs=[
            pl.BlockSpec((BLOCK_E, N_GENES, N_CELLS), lambda i: (i, 0, 0)),
            pl.BlockSpec(memory_space=pltpu.SMEM),
        ],
        out_specs=pl.BlockSpec((N_GENES, N_CELLS), lambda i: (0, 0)),
        out_shape=jax.ShapeDtypeStruct((N_GENES, N_CELLS), jnp.float32),
    )(xt, w2)


def _bias_body(p_ref, b_ref, o_ref):
    b = b_ref[...].reshape(N_GENES, 1)
    o_ref[...] = p_ref[...] + b


def _tc_bias_add(partial_t, bias_g):
    return pl.pallas_call(
        _bias_body,
        in_specs=[
            pl.BlockSpec((N_GENES, N_CELLS), lambda: (0, 0)),
            pl.BlockSpec((N_GENES,), lambda: (0,)),
        ],
        out_specs=pl.BlockSpec((N_GENES, N_CELLS), lambda: (0, 0)),
        out_shape=jax.ShapeDtypeStruct((N_GENES, N_CELLS), jnp.float32),
    )(partial_t, bias_g)


def kernel(cell_gene_embedding, gene_ix, weight1, bias1):
    xt = jnp.transpose(cell_gene_embedding, (2, 1, 0))
    partial_t = _tc_matvec(xt, weight1.reshape(N_EMB, 1))
    bias_g = _sc_gather(bias1, gene_ix.astype(jnp.int32))
    out_t = _tc_bias_add(partial_t, bias_g)
    return out_t.T
